# baseline (speedup 1.0000x reference)
import dataclasses

import jax
import jax.numpy as jnp
from jax import lax
from jax.experimental import pallas as pl
from jax.experimental.pallas import tpu as pltpu
from jax.experimental.pallas import tpu_sc as plsc

N = 10000
D = 128
C = 16
NC = 2
NS = 16
NW = NC * NS
CHUNK = 128
N_PAD = 10240
E_PAD = 327680
EPW = E_PAD // NW
NCHUNK = EPW // CHUNK
ROWS_PS = N_PAD // NS

_mesh = plsc.VectorSubcoreMesh(core_axis_name="c", subcore_axis_name="s")


_PHASES = 2
_CPP = NCHUNK // _PHASES

_AGG_OUT_TYPE = jax.ShapeDtypeStruct((NC, N_PAD, D), jnp.float32)
_AGG_SCRATCH = [
    pltpu.VMEM((_CPP, CHUNK), jnp.int32),
    pltpu.VMEM((_CPP, CHUNK), jnp.int32),
    pltpu.VMEM((CHUNK, D), jnp.float32),
    pltpu.VMEM((CHUNK, D), jnp.float32),
    pltpu.VMEM_SHARED((N_PAD, D), jnp.float32),
    pltpu.SemaphoreType.DMA,
    pltpu.SemaphoreType.DMA,
    pltpu.SemaphoreType.DMA,
    pltpu.SemaphoreType.DMA,
]


def _sc_aggregate_body(table_hbm, src_hbm, dst_hbm, zeros_hbm,
                       out_hbm,
                       sidx_v, didx_v, rows0_v, rows1_v, acc_sh,
                       gsem0, gsem1, ssem0, ssem1):
    cid = lax.axis_index("c")
    sid = lax.axis_index("s")
    wid = cid * NS + sid
    r0 = sid * ROWS_PS
    rows = (rows0_v, rows1_v)
    gsems = (gsem0, gsem1)
    ssems = (ssem0, ssem1)

    pltpu.sync_copy(zeros_hbm, acc_sh.at[pl.ds(r0, ROWS_PS)])
    plsc.subcore_barrier()

    def _gather_start(c, b):
        pltpu.async_copy(table_hbm.at[sidx_v.at[c]], rows[b], gsems[b])

    def _gather_wait(c, b):
        pltpu.make_async_copy(table_hbm.at[sidx_v.at[c]], rows[b],
                              gsems[b]).wait()

    def _scatter_start(c, b):
        pltpu.async_copy(rows[b], acc_sh.at[didx_v.at[c]], ssems[b],
                         add=True)

    def _scatter_wait(c, b):
        pltpu.make_async_copy(rows[b], acc_sh.at[didx_v.at[c]],
                              ssems[b]).wait()

    for p in range(_PHASES):
        pltpu.sync_copy(src_hbm.at[wid, pl.ds(p * _CPP, _CPP)], sidx_v)
        pltpu.sync_copy(dst_hbm.at[wid, pl.ds(p * _CPP, _CPP)], didx_v)
        _gather_start(0, 0)

        @pl.loop(0, _CPP, step=2)
        def _(i):
            for b in range(2):
                cur = i + b

                @pl.when(cur + 1 < _CPP)
                def _():
                    @pl.when(cur >= 1)
                    def _():
                        _scatter_wait(cur - 1, 1 - b)

                    _gather_start(cur + 1, 1 - b)

                _gather_wait(cur, b)
                _scatter_start(cur, b)

        _scatter_wait(_CPP - 1, (_CPP - 1) % 2)

    plsc.subcore_barrier()
    pltpu.sync_copy(acc_sh.at[pl.ds(r0, ROWS_PS)],
                    out_hbm.at[cid, pl.ds(r0, ROWS_PS)])


_sc_aggregate = pl.kernel(
    _sc_aggregate_body, out_type=_AGG_OUT_TYPE, mesh=_mesh,
    scratch_types=_AGG_SCRATCH)


_cp = pltpu.CompilerParams()
if "needs_layout_passes" in pltpu.CompilerParams.__dataclass_fields__:
    _cp = dataclasses.replace(_cp, needs_layout_passes=False)

_HIST_OUT_TYPE = (
    jax.ShapeDtypeStruct((NW, N_PAD), jnp.float32),
    jax.ShapeDtypeStruct((NW, N_PAD), jnp.float32),
)
_HIST_SCRATCH = [
    pltpu.VMEM((NCHUNK, CHUNK), jnp.int32),
    pltpu.VMEM((NCHUNK, CHUNK), jnp.int32),
    pltpu.VMEM((N_PAD,), jnp.float32),
    pltpu.VMEM((N_PAD,), jnp.float32),
]


def _sc_hist_body(src_hbm, dst_hbm, outs_hbm, outd_hbm,
                  sidx_v, didx_v, hs_v, hd_v):
    cid = lax.axis_index("c")
    sid = lax.axis_index("s")
    wid = cid * NS + sid
    ones16 = jnp.ones((16,), jnp.float32)
    zeros16 = jnp.zeros((16,), jnp.float32)

    @pl.loop(0, N_PAD, step=16)
    def _(i):
        hs_v[pl.ds(i, 16)] = zeros16
        hd_v[pl.ds(i, 16)] = zeros16

    pltpu.sync_copy(src_hbm.at[wid], sidx_v)
    pltpu.sync_copy(dst_hbm.at[wid], didx_v)

    @pl.loop(0, NCHUNK)
    def _(c):
        for j in range(8):
            iv = sidx_v[c, pl.ds(16 * j, 16)]
            plsc.addupdate_scatter(hs_v, [iv], ones16)
            dv = didx_v[c, pl.ds(16 * j, 16)]
            plsc.addupdate_scatter(hd_v, [dv], ones16)

    pltpu.sync_copy(hs_v, outs_hbm.at[wid])
    pltpu.sync_copy(hd_v, outd_hbm.at[wid])


_sc_hist = pl.kernel(
    _sc_hist_body, out_type=_HIST_OUT_TYPE, mesh=_mesh,
    scratch_types=_HIST_SCRATCH, compiler_params=_cp)


def _tc_deg_body(hs_ref, hd_ref, do_ref, di_ref):
    deg_out = jnp.sum(hs_ref[...], axis=0, keepdims=True)
    deg_in = jnp.sum(hd_ref[...], axis=0, keepdims=True)
    do_ref[...] = lax.rsqrt(jnp.maximum(deg_out, 1.0))
    di_ref[...] = lax.rsqrt(jnp.maximum(deg_in, 1.0))


def _tc1_body(x_ref, do_ref, w1_ref, hw_ref):
    hw_ref[...] = jnp.dot(x_ref[...] * do_ref[...], w1_ref[...],
                          preferred_element_type=jnp.float32)


def _tc2_body(agg_ref, di_ref, do_ref, b_ref, w_ref, hw_ref):
    agg = agg_ref[0] + agg_ref[1]
    h = agg * di_ref[...] + b_ref[...]
    h = jnp.where(h >= 0.0, h, 0.01 * h)
    hw_ref[...] = jnp.dot(h * do_ref[...], w_ref[...],
                          preferred_element_type=jnp.float32)


def _tc3_body(agg_ref, di_ref, b_ref, wl_ref, bl_ref, out_ref):
    agg = agg_ref[0] + agg_ref[1]
    h = agg * di_ref[...] + b_ref[...]
    h = jnp.where(h >= 0.0, h, 0.01 * h)
    out_ref[...] = jnp.dot(h, wl_ref[...],
                           preferred_element_type=jnp.float32) + bl_ref[...]


def kernel(x, edge_index, W1, b1, W2, b2, Wl, bl):
    src = edge_index[0].astype(jnp.int32)
    dst = edge_index[1].astype(jnp.int32)
    n_extra = E_PAD - src.shape[0]
    pad_idx = N + (jnp.arange(n_extra, dtype=jnp.int32) % (N_PAD - N))
    src = jnp.concatenate([src, pad_idx]).reshape(NW, NCHUNK, CHUNK)
    dst = jnp.concatenate([dst, pad_idx]).reshape(NW, NCHUNK, CHUNK)

    x_pad = jnp.concatenate(
        [x, jnp.zeros((N_PAD - N, D), jnp.float32)], axis=0)

    zerosD = jnp.zeros((ROWS_PS, D), jnp.float32)

    hist_s, hist_d = _sc_hist(src, dst)

    do_row, di_row = pl.pallas_call(
        _tc_deg_body,
        out_shape=(
            jax.ShapeDtypeStruct((1, N_PAD), jnp.float32),
            jax.ShapeDtypeStruct((1, N_PAD), jnp.float32),
        ),
    )(hist_s, hist_d)
    do = do_row.reshape(N_PAD, 1)
    di = di_row.reshape(N_PAD, 1)

    hw1 = pl.pallas_call(
        _tc1_body,
        out_shape=jax.ShapeDtypeStruct((N_PAD, D), jnp.float32),
    )(x_pad, do, W1)

    agg1 = _sc_aggregate(hw1, src, dst, zerosD)

    hw2 = pl.pallas_call(
        _tc2_body,
        out_shape=jax.ShapeDtypeStruct((N_PAD, D), jnp.float32),
    )(agg1, di, do, b1.reshape(1, D), W2)

    agg2 = _sc_aggregate(hw2, src, dst, zerosD)

    out = pl.pallas_call(
        _tc3_body,
        out_shape=jax.ShapeDtypeStruct((N_PAD, C), jnp.float32),
    )(agg2, di, b2.reshape(1, D), Wl, bl.reshape(1, C))

    return out[:N]


# speedup vs baseline: 1.1384x; 1.1384x over previous
import dataclasses

import jax
import jax.numpy as jnp
from jax import lax
from jax.experimental import pallas as pl
from jax.experimental.pallas import tpu as pltpu
from jax.experimental.pallas import tpu_sc as plsc

N = 10000
D = 128
C = 16
NC = 2
NS = 16
NW = NC * NS
CHUNK = 128
N_PAD = 10240
E_PAD = 327680
EPW = E_PAD // NW
NCHUNK = EPW // CHUNK
ROWS_PS = N_PAD // NS

_mesh = plsc.VectorSubcoreMesh(core_axis_name="c", subcore_axis_name="s")


_PHASES = 2
_CPP = NCHUNK // _PHASES

_AGG_OUT_TYPE = jax.ShapeDtypeStruct((NC, N_PAD, D), jnp.float32)
_AGG_SCRATCH = [
    pltpu.VMEM((_CPP, CHUNK), jnp.int32),
    pltpu.VMEM((_CPP, CHUNK), jnp.int32),
    pltpu.VMEM((CHUNK, D), jnp.float32),
    pltpu.VMEM((CHUNK, D), jnp.float32),
    pltpu.VMEM_SHARED((N_PAD, D), jnp.float32),
    pltpu.SemaphoreType.DMA,
    pltpu.SemaphoreType.DMA,
]


def _sc_aggregate_body(table_hbm, src_hbm, dst_hbm, zeros_hbm,
                       out_hbm,
                       sidx_v, didx_v, rows0_v, rows1_v, acc_sh, sem0, sem1):
    cid = lax.axis_index("c")
    sid = lax.axis_index("s")
    wid = cid * NS + sid
    r0 = sid * ROWS_PS
    rows = (rows0_v, rows1_v)
    sems = (sem0, sem1)

    pltpu.sync_copy(zeros_hbm, acc_sh.at[pl.ds(r0, ROWS_PS)])
    plsc.subcore_barrier()

    def _gather_start(c, b):
        pltpu.async_copy(table_hbm.at[sidx_v.at[c]], rows[b], sems[b])

    def _gather_wait(c, b):
        pltpu.make_async_copy(table_hbm.at[sidx_v.at[c]], rows[b],
                              sems[b]).wait()

    for p in range(_PHASES):
        pltpu.sync_copy(src_hbm.at[wid, pl.ds(p * _CPP, _CPP)], sidx_v)
        pltpu.sync_copy(dst_hbm.at[wid, pl.ds(p * _CPP, _CPP)], didx_v)
        _gather_start(0, 0)

        @pl.loop(0, _CPP, step=2)
        def _(i):
            for b in range(2):
                cur = i + b

                @pl.when(cur + 1 < _CPP)
                def _():
                    _gather_start(cur + 1, 1 - b)

                _gather_wait(cur, b)
                pltpu.sync_copy(rows[b], acc_sh.at[didx_v.at[cur]], add=True)

    plsc.subcore_barrier()
    pltpu.sync_copy(acc_sh.at[pl.ds(r0, ROWS_PS)],
                    out_hbm.at[cid, pl.ds(r0, ROWS_PS)])


_sc_aggregate = pl.kernel(
    _sc_aggregate_body, out_type=_AGG_OUT_TYPE, mesh=_mesh,
    scratch_types=_AGG_SCRATCH)


_cp = pltpu.CompilerParams()
if "needs_layout_passes" in pltpu.CompilerParams.__dataclass_fields__:
    _cp = dataclasses.replace(_cp, needs_layout_passes=False)

_HIST_OUT_TYPE = (
    jax.ShapeDtypeStruct((NW, N_PAD), jnp.float32),
    jax.ShapeDtypeStruct((NW, N_PAD), jnp.float32),
)
_HIST_SCRATCH = [
    pltpu.VMEM((NCHUNK, CHUNK), jnp.int32),
    pltpu.VMEM((NCHUNK, CHUNK), jnp.int32),
    pltpu.VMEM((N_PAD,), jnp.float32),
    pltpu.VMEM((N_PAD,), jnp.float32),
]


def _sc_hist_body(src_hbm, dst_hbm, outs_hbm, outd_hbm,
                  sidx_v, didx_v, hs_v, hd_v):
    cid = lax.axis_index("c")
    sid = lax.axis_index("s")
    wid = cid * NS + sid
    ones16 = jnp.ones((16,), jnp.float32)
    zeros16 = jnp.zeros((16,), jnp.float32)

    @pl.loop(0, N_PAD, step=16)
    def _(i):
        hs_v[pl.ds(i, 16)] = zeros16
        hd_v[pl.ds(i, 16)] = zeros16

    pltpu.sync_copy(src_hbm.at[wid], sidx_v)
    pltpu.sync_copy(dst_hbm.at[wid], didx_v)

    @pl.loop(0, NCHUNK)
    def _(c):
        for j in range(8):
            iv = sidx_v[c, pl.ds(16 * j, 16)]
            plsc.addupdate_scatter(hs_v, [iv], ones16)
            dv = didx_v[c, pl.ds(16 * j, 16)]
            plsc.addupdate_scatter(hd_v, [dv], ones16)

    pltpu.sync_copy(hs_v, outs_hbm.at[wid])
    pltpu.sync_copy(hd_v, outd_hbm.at[wid])


_sc_hist = pl.kernel(
    _sc_hist_body, out_type=_HIST_OUT_TYPE, mesh=_mesh,
    scratch_types=_HIST_SCRATCH, compiler_params=_cp)


def _tc_deg_body(hs_ref, hd_ref, do_ref, di_ref):
    deg_out = jnp.sum(hs_ref[...], axis=0, keepdims=True)
    deg_in = jnp.sum(hd_ref[...], axis=0, keepdims=True)
    do_ref[...] = lax.rsqrt(jnp.maximum(deg_out, 1.0))
    di_ref[...] = lax.rsqrt(jnp.maximum(deg_in, 1.0))


def _tc1_body(x_ref, do_ref, w1_ref, hw_ref):
    hw_ref[...] = jnp.dot(x_ref[...] * do_ref[...], w1_ref[...],
                          preferred_element_type=jnp.float32)


def _tc2_body(agg_ref, di_ref, do_ref, b_ref, w_ref, hw_ref):
    agg = agg_ref[0] + agg_ref[1]
    h = agg * di_ref[...] + b_ref[...]
    h = jnp.where(h >= 0.0, h, 0.01 * h)
    hw_ref[...] = jnp.dot(h * do_ref[...], w_ref[...],
                          preferred_element_type=jnp.float32)


def _tc3_body(agg_ref, di_ref, b_ref, wl_ref, bl_ref, out_ref):
    agg = agg_ref[0] + agg_ref[1]
    h = agg * di_ref[...] + b_ref[...]
    h = jnp.where(h >= 0.0, h, 0.01 * h)
    out_ref[...] = jnp.dot(h, wl_ref[...],
                           preferred_element_type=jnp.float32) + bl_ref[...]


def kernel(x, edge_index, W1, b1, W2, b2, Wl, bl):
    src = edge_index[0].astype(jnp.int32)
    dst = edge_index[1].astype(jnp.int32)
    n_extra = E_PAD - src.shape[0]
    pad_idx = N + (jnp.arange(n_extra, dtype=jnp.int32) % (N_PAD - N))
    src = jnp.concatenate([src, pad_idx]).reshape(NW, NCHUNK, CHUNK)
    dst = jnp.concatenate([dst, pad_idx]).reshape(NW, NCHUNK, CHUNK)

    x_pad = jnp.concatenate(
        [x, jnp.zeros((N_PAD - N, D), jnp.float32)], axis=0)

    zerosD = jnp.zeros((ROWS_PS, D), jnp.float32)

    hist_s, hist_d = _sc_hist(src, dst)

    do_row, di_row = pl.pallas_call(
        _tc_deg_body,
        out_shape=(
            jax.ShapeDtypeStruct((1, N_PAD), jnp.float32),
            jax.ShapeDtypeStruct((1, N_PAD), jnp.float32),
        ),
    )(hist_s, hist_d)
    do = do_row.reshape(N_PAD, 1)
    di = di_row.reshape(N_PAD, 1)

    hw1 = pl.pallas_call(
        _tc1_body,
        out_shape=jax.ShapeDtypeStruct((N_PAD, D), jnp.float32),
    )(x_pad, do, W1)

    agg1 = _sc_aggregate(hw1, src, dst, zerosD)

    hw2 = pl.pallas_call(
        _tc2_body,
        out_shape=jax.ShapeDtypeStruct((N_PAD, D), jnp.float32),
    )(agg1, di, do, b1.reshape(1, D), W2)

    agg2 = _sc_aggregate(hw2, src, dst, zerosD)

    out = pl.pallas_call(
        _tc3_body,
        out_shape=jax.ShapeDtypeStruct((N_PAD, C), jnp.float32),
    )(agg2, di, b2.reshape(1, D), Wl, bl.reshape(1, C))

    return out[:N]


# speedup vs baseline: 1.1626x; 1.0213x over previous
import dataclasses

import jax
import jax.numpy as jnp
from jax import lax
from jax.experimental import pallas as pl
from jax.experimental.pallas import tpu as pltpu
from jax.experimental.pallas import tpu_sc as plsc

N = 10000
D = 128
C = 16
NC = 2
NS = 16
NW = NC * NS
CHUNK = 128
N_PAD = 10240
E_PAD = 327680
EPW = E_PAD // NW
NCHUNK = EPW // CHUNK
ROWS_PS = N_PAD // NS

_mesh = plsc.VectorSubcoreMesh(core_axis_name="c", subcore_axis_name="s")


_PHASES = 2
_CPP = NCHUNK // _PHASES

_AGG_OUT_TYPE = jax.ShapeDtypeStruct((NC, N_PAD, D), jnp.float32)
_AGG_SCRATCH = [
    pltpu.VMEM((_CPP, CHUNK), jnp.int32),
    pltpu.VMEM((_CPP, CHUNK), jnp.int32),
    pltpu.VMEM((CHUNK, D), jnp.float32),
    pltpu.VMEM((CHUNK, D), jnp.float32),
    pltpu.VMEM_SHARED((N_PAD, D), jnp.float32),
    pltpu.SemaphoreType.DMA,
    pltpu.SemaphoreType.DMA,
]


def _sc_aggregate_body(table_hbm, src_hbm, dst_hbm, zeros_hbm,
                       out_hbm,
                       sidx_v, didx_v, rows0_v, rows1_v, acc_sh, sem0, sem1):
    cid = lax.axis_index("c")
    sid = lax.axis_index("s")
    wid = cid * NS + sid
    r0 = sid * ROWS_PS
    rows = (rows0_v, rows1_v)
    sems = (sem0, sem1)

    pltpu.sync_copy(zeros_hbm, acc_sh.at[pl.ds(r0, ROWS_PS)])
    plsc.subcore_barrier()

    def _gather_start(c, b):
        pltpu.async_copy(table_hbm.at[sidx_v.at[c]], rows[b], sems[b])

    def _gather_wait(c, b):
        pltpu.make_async_copy(table_hbm.at[sidx_v.at[c]], rows[b],
                              sems[b]).wait()

    for p in range(_PHASES):
        pltpu.sync_copy(src_hbm.at[wid, pl.ds(p * _CPP, _CPP)], sidx_v)
        pltpu.sync_copy(dst_hbm.at[wid, pl.ds(p * _CPP, _CPP)], didx_v)
        _gather_start(0, 0)

        @pl.loop(0, _CPP, step=2)
        def _(i):
            for b in range(2):
                cur = i + b

                @pl.when(cur + 1 < _CPP)
                def _():
                    _gather_start(cur + 1, 1 - b)

                _gather_wait(cur, b)
                pltpu.sync_copy(rows[b], acc_sh.at[didx_v.at[cur]], add=True)

    plsc.subcore_barrier()
    pltpu.sync_copy(acc_sh.at[pl.ds(r0, ROWS_PS)],
                    out_hbm.at[cid, pl.ds(r0, ROWS_PS)])


_sc_aggregate = pl.kernel(
    _sc_aggregate_body, out_type=_AGG_OUT_TYPE, mesh=_mesh,
    scratch_types=_AGG_SCRATCH)


_cp = pltpu.CompilerParams()
if "needs_layout_passes" in pltpu.CompilerParams.__dataclass_fields__:
    _cp = dataclasses.replace(_cp, needs_layout_passes=False)

_HIST_OUT_TYPE = (
    jax.ShapeDtypeStruct((NW, N_PAD), jnp.float32),
    jax.ShapeDtypeStruct((NW, N_PAD), jnp.float32),
)
_HIST_SCRATCH = [
    pltpu.VMEM((NCHUNK, CHUNK), jnp.int32),
    pltpu.VMEM((NCHUNK, CHUNK), jnp.int32),
    pltpu.VMEM((N_PAD,), jnp.float32),
    pltpu.VMEM((N_PAD,), jnp.float32),
]


def _sc_hist_body(src_hbm, dst_hbm, outs_hbm, outd_hbm,
                  sidx_v, didx_v, hs_v, hd_v):
    cid = lax.axis_index("c")
    sid = lax.axis_index("s")
    wid = cid * NS + sid
    ones16 = jnp.ones((16,), jnp.float32)
    zeros16 = jnp.zeros((16,), jnp.float32)

    @pl.loop(0, N_PAD, step=16)
    def _(i):
        hs_v[pl.ds(i, 16)] = zeros16
        hd_v[pl.ds(i, 16)] = zeros16

    pltpu.sync_copy(src_hbm.at[wid], sidx_v)
    pltpu.sync_copy(dst_hbm.at[wid], didx_v)

    @pl.loop(0, NCHUNK)
    def _(c):
        for j in range(8):
            iv = sidx_v[c, pl.ds(16 * j, 16)]
            plsc.addupdate_scatter(hs_v, [iv], ones16)
            dv = didx_v[c, pl.ds(16 * j, 16)]
            plsc.addupdate_scatter(hd_v, [dv], ones16)

    pltpu.sync_copy(hs_v, outs_hbm.at[wid])
    pltpu.sync_copy(hd_v, outd_hbm.at[wid])


_sc_hist = pl.kernel(
    _sc_hist_body, out_type=_HIST_OUT_TYPE, mesh=_mesh,
    scratch_types=_HIST_SCRATCH, compiler_params=_cp)


def _tc1_body(hs_ref, hd_ref, x_ref, w1_ref, hw_ref, do_ref, di_ref):
    deg_out = jnp.sum(hs_ref[...], axis=0, keepdims=True)
    deg_in = jnp.sum(hd_ref[...], axis=0, keepdims=True)
    do = lax.rsqrt(jnp.maximum(deg_out, 1.0)).reshape(N_PAD, 1)
    di = lax.rsqrt(jnp.maximum(deg_in, 1.0)).reshape(N_PAD, 1)
    do_ref[...] = do
    di_ref[...] = di
    hw_ref[...] = jnp.dot(x_ref[...] * do, w1_ref[...],
                          preferred_element_type=jnp.float32)


def _tc2_body(agg_ref, di_ref, do_ref, b_ref, w_ref, hw_ref):
    agg = agg_ref[0] + agg_ref[1]
    h = agg * di_ref[...] + b_ref[...]
    h = jnp.where(h >= 0.0, h, 0.01 * h)
    hw_ref[...] = jnp.dot(h * do_ref[...], w_ref[...],
                          preferred_element_type=jnp.float32)


def _tc3_body(agg_ref, di_ref, b_ref, wl_ref, bl_ref, out_ref):
    agg = agg_ref[0] + agg_ref[1]
    h = agg * di_ref[...] + b_ref[...]
    h = jnp.where(h >= 0.0, h, 0.01 * h)
    out_ref[...] = jnp.dot(h, wl_ref[...],
                           preferred_element_type=jnp.float32) + bl_ref[...]


def kernel(x, edge_index, W1, b1, W2, b2, Wl, bl):
    src = edge_index[0].astype(jnp.int32)
    dst = edge_index[1].astype(jnp.int32)
    n_extra = E_PAD - src.shape[0]
    pad_idx = N + (jnp.arange(n_extra, dtype=jnp.int32) % (N_PAD - N))
    src = jnp.concatenate([src, pad_idx]).reshape(NW, NCHUNK, CHUNK)
    dst = jnp.concatenate([dst, pad_idx]).reshape(NW, NCHUNK, CHUNK)

    x_pad = jnp.concatenate(
        [x, jnp.zeros((N_PAD - N, D), jnp.float32)], axis=0)

    zerosD = jnp.zeros((ROWS_PS, D), jnp.float32)

    hist_s, hist_d = _sc_hist(src, dst)

    hw1, do, di = pl.pallas_call(
        _tc1_body,
        out_shape=(
            jax.ShapeDtypeStruct((N_PAD, D), jnp.float32),
            jax.ShapeDtypeStruct((N_PAD, 1), jnp.float32),
            jax.ShapeDtypeStruct((N_PAD, 1), jnp.float32),
        ),
    )(hist_s, hist_d, x_pad, W1)

    agg1 = _sc_aggregate(hw1, src, dst, zerosD)

    hw2 = pl.pallas_call(
        _tc2_body,
        out_shape=jax.ShapeDtypeStruct((N_PAD, D), jnp.float32),
    )(agg1, di, do, b1.reshape(1, D), W2)

    agg2 = _sc_aggregate(hw2, src, dst, zerosD)

    out = pl.pallas_call(
        _tc3_body,
        out_shape=jax.ShapeDtypeStruct((N_PAD, C), jnp.float32),
    )(agg2, di, b2.reshape(1, D), Wl, bl.reshape(1, C))

    return out[:N]
